# baseline (speedup 1.0000x reference)
import functools

import jax
import jax.numpy as jnp
from jax import lax
from jax.experimental import pallas as pl
from jax.experimental.pallas import tpu as pltpu
from jax.experimental.pallas import tpu_sc as plsc

H_DIM = 32
NUM_CORES = 2
NUM_SUBCORES = 16
NUM_WORKERS = NUM_CORES * NUM_SUBCORES
CHUNK = 1600


def _build_gather(total_rows: int):
    rows_per_worker = total_rows // NUM_WORKERS
    num_steps = rows_per_worker // CHUNK
    assert rows_per_worker % CHUNK == 0

    mesh = plsc.VectorSubcoreMesh(core_axis_name="c", subcore_axis_name="s")

    @functools.partial(
        pl.kernel,
        mesh=mesh,
        out_type=jax.ShapeDtypeStruct((total_rows, H_DIM), jnp.float32),
        scratch_types=[
            pltpu.VMEM((CHUNK,), jnp.int32),
            pltpu.VMEM((CHUNK,), jnp.int32),
            pltpu.VMEM((CHUNK, H_DIM), jnp.float32),
            pltpu.VMEM((CHUNK, H_DIM), jnp.float32),
            pltpu.SemaphoreType.DMA,
            pltpu.SemaphoreType.DMA,
            pltpu.SemaphoreType.DMA,
            pltpu.SemaphoreType.DMA,
        ],
        compiler_params=pltpu.CompilerParams(use_tc_tiling_on_sc=False),
    )
    def gather_kernel(idx_hbm, table_hbm, out_hbm,
                      idx_v0, idx_v1, rows_v0, rows_v1,
                      sem_g0, sem_g1, sem_o0, sem_o1):
        wid = lax.axis_index("s") * NUM_CORES + lax.axis_index("c")
        base = wid * rows_per_worker

        idx_v = (idx_v0, idx_v1)
        rows_v = (rows_v0, rows_v1)
        sem_g = (sem_g0, sem_g1)
        sem_o = (sem_o0, sem_o1)

        gath = [None, None]
        wb = [None, None]
        for s in range(num_steps):
            b = s % 2
            if wb[b] is not None:
                wb[b].wait()
                wb[b] = None
            off = base + s * CHUNK
            pltpu.sync_copy(idx_hbm.at[pl.ds(off, CHUNK)], idx_v[b])
            gath[b] = pltpu.async_copy(
                table_hbm.at[idx_v[b]], rows_v[b], sem_g[b])
            if s > 0:
                pb = 1 - b
                gath[pb].wait()
                gath[pb] = None
                poff = base + (s - 1) * CHUNK
                wb[pb] = pltpu.async_copy(
                    rows_v[pb], out_hbm.at[pl.ds(poff, CHUNK)], sem_o[pb])
        bl = (num_steps - 1) % 2
        gath[bl].wait()
        loff = base + (num_steps - 1) * CHUNK
        wb[bl] = pltpu.async_copy(
            rows_v[bl], out_hbm.at[pl.ds(loff, CHUNK)], sem_o[bl])
        wb[0].wait()
        wb[1].wait()

    return gather_kernel


def kernel(inputs, emb_weight):
    flat_idx = inputs.reshape(-1).astype(jnp.int32)
    gather = _build_gather(flat_idx.shape[0])
    out = gather(flat_idx, emb_weight)
    return out.reshape(inputs.shape + (emb_weight.shape[1],))
